# baseline (speedup 1.0000x reference)
import jax
import jax.numpy as jnp
from jax import lax
from jax.experimental import pallas as pl
from jax.experimental.pallas import tpu as pltpu
from jax.experimental.pallas import tpu_sc as plsc

N_NODES = 10000
D = 128
B = 320000

_INFO = plsc.get_sparse_core_info()
NC = _INFO.num_cores
NS = _INFO.num_subcores
NW = NC * NS
L = _INFO.num_lanes

EDGES_PER_W = B // NW
CHUNK = 64
ROWS = 2 * CHUNK
N_CHUNKS = EDGES_PER_W // CHUNK
TAIL = EDGES_PER_W - N_CHUNKS * CHUNK
N_PAIRS = N_CHUNKS // 2
GROUPS = CHUNK // L
STRIPE = 640


def _body(z_hbm, srci_hbm, dsti_hbm, out_hbm,
          ib0, ib1, rb0, rb1, ob0, ob1, jvecs,
          zsh, sem0, sem1, isem0, isem1, osem0, osem1):
    cid = lax.axis_index("c")
    sid = lax.axis_index("s")
    wid = sid * NC + cid
    base = wid * EDGES_PER_W

    zoff = jnp.minimum(sid * STRIPE, N_NODES - STRIPE)
    pltpu.sync_copy(z_hbm.at[pl.ds(zoff, STRIPE)], zsh.at[pl.ds(zoff, STRIPE)])
    plsc.subcore_barrier()

    lanes = lax.iota(jnp.int32, L)
    lane_base = lanes * D

    def build_jvec(j, carry):
        jvecs[j] = lane_base + ((j + lanes) & (D - 1))
        return carry

    lax.fori_loop(0, D, build_jvec, 0)
    zero16 = jnp.zeros((L,), jnp.int32)

    def issue_idx(c, ib, isem):
        pltpu.async_copy(srci_hbm.at[pl.ds(base + c * CHUNK, CHUNK)],
                         ib.at[pl.ds(0, CHUNK)], isem)
        pltpu.async_copy(dsti_hbm.at[pl.ds(base + c * CHUNK, CHUNK)],
                         ib.at[pl.ds(CHUNK, CHUNK)], isem)

    def wait_idx(ib, isem):
        pltpu.make_async_copy(srci_hbm.at[pl.ds(0, CHUNK)],
                              ib.at[pl.ds(0, CHUNK)], isem).wait()
        pltpu.make_async_copy(srci_hbm.at[pl.ds(0, CHUNK)],
                              ib.at[pl.ds(CHUNK, CHUNK)], isem).wait()

    def issue_rows(ib, rb, sem):
        pltpu.async_copy(zsh.at[ib], rb, sem)

    def wait_rows(ib, rb, sem):
        pltpu.make_async_copy(zsh.at[ib], rb, sem).wait()

    def wait_out(ob, osem):
        pltpu.make_async_copy(ob, out_hbm.at[pl.ds(0, CHUNK)], osem).wait()

    def compute(c, rb, ob, osem, ngroups=GROUPS):
        zero = jnp.zeros((L,), jnp.float32)
        JBLK = 8
        DOFF = CHUNK * D

        def block(b, accs):
            accs = list(accs)
            for jj in range(JBLK):
                jv = jvecs[b * JBLK + jj]
                for g in range(ngroups):
                    fs = (jv + (g * L * D)) if g else jv
                    sv = plsc.load_gather(rb, [zero16, fs])
                    dv = plsc.load_gather(rb, [zero16, fs + DOFF])
                    accs[g] = accs[g] + sv * dv
            return tuple(accs)

        accs = lax.fori_loop(0, D // JBLK, block, (zero,) * ngroups)
        for g in range(ngroups):
            res = 1.0 / (1.0 + jnp.exp(-accs[g]))
            ob[pl.ds(g * L, L)] = res
        pltpu.async_copy(ob.at[pl.ds(0, ngroups * L)],
                         out_hbm.at[pl.ds(base + c * CHUNK, ngroups * L)], osem)

    issue_idx(0, ib0, isem0)
    issue_idx(1, ib1, isem1)
    wait_idx(ib0, isem0)
    issue_rows(ib0, rb0, sem0)

    def pair(i, carry):
        c0 = 2 * i
        c1 = 2 * i + 1

        wait_idx(ib1, isem1)
        issue_rows(ib1, rb1, sem1)

        wait_rows(ib0, rb0, sem0)

        @pl.when(c0 + 2 < N_CHUNKS)
        def _():
            issue_idx(c0 + 2, ib0, isem0)

        @pl.when(i > 0)
        def _():
            wait_out(ob0, osem0)

        compute(c0, rb0, ob0, osem0)

        @pl.when(c0 + 2 < N_CHUNKS)
        def _():
            wait_idx(ib0, isem0)
            issue_rows(ib0, rb0, sem0)

        wait_rows(ib1, rb1, sem1)

        @pl.when(c1 + 2 < N_CHUNKS)
        def _():
            issue_idx(c1 + 2, ib1, isem1)

        @pl.when(i > 0)
        def _():
            wait_out(ob1, osem1)

        compute(c1, rb1, ob1, osem1)
        return carry

    lax.fori_loop(0, N_PAIRS, pair, 0)

    pltpu.async_copy(srci_hbm.at[pl.ds(base + N_CHUNKS * CHUNK, TAIL)],
                     ib0.at[pl.ds(0, TAIL)], isem0).wait()
    pltpu.async_copy(dsti_hbm.at[pl.ds(base + N_CHUNKS * CHUNK, TAIL)],
                     ib0.at[pl.ds(TAIL, TAIL)], isem0).wait()
    pltpu.async_copy(zsh.at[ib0.at[pl.ds(0, 2 * TAIL)]],
                     rb0.at[pl.ds(0, 2 * TAIL)], sem0).wait()
    wait_out(ob0, osem0)

    zero = jnp.zeros((L,), jnp.float32)

    def tail_block(b, acc):
        acc2 = acc
        for jj in range(8):
            jv = jvecs[b * 8 + jj]
            sv = plsc.load_gather(rb0, [zero16, jv])
            dv = plsc.load_gather(rb0, [zero16, jv + TAIL * D])
            acc2 = acc2 + sv * dv
        return acc2

    tacc = lax.fori_loop(0, D // 8, tail_block, zero)
    ob0[pl.ds(0, L)] = 1.0 / (1.0 + jnp.exp(-tacc))
    pltpu.async_copy(ob0.at[pl.ds(0, TAIL)],
                     out_hbm.at[pl.ds(base + N_CHUNKS * CHUNK, TAIL)], osem0)
    wait_out(ob1, osem1)
    pltpu.make_async_copy(ob0.at[pl.ds(0, TAIL)],
                          out_hbm.at[pl.ds(0, TAIL)], osem0).wait()


@jax.jit
def _run(z, src, dst):
    mesh = plsc.VectorSubcoreMesh(core_axis_name="c", subcore_axis_name="s")
    k = pl.kernel(
        _body,
        mesh=mesh,
        compiler_params=pltpu.CompilerParams(needs_layout_passes=False),
        out_type=jax.ShapeDtypeStruct((B,), jnp.float32),
        scratch_types=[
            pltpu.VMEM((ROWS,), jnp.int32),
            pltpu.VMEM((ROWS,), jnp.int32),
            pltpu.VMEM((ROWS, D), jnp.float32),
            pltpu.VMEM((ROWS, D), jnp.float32),
            pltpu.VMEM((CHUNK,), jnp.float32),
            pltpu.VMEM((CHUNK,), jnp.float32),
            pltpu.VMEM((D, L), jnp.int32),
            pltpu.VMEM_SHARED((N_NODES, D), jnp.float32),
            pltpu.SemaphoreType.DMA,
            pltpu.SemaphoreType.DMA,
            pltpu.SemaphoreType.DMA,
            pltpu.SemaphoreType.DMA,
            pltpu.SemaphoreType.DMA,
            pltpu.SemaphoreType.DMA,
        ],
    )
    return k(z, src, dst)


def kernel(z, edge_index):
    src = edge_index[0].astype(jnp.int32)
    dst = edge_index[1].astype(jnp.int32)
    return _run(z, src, dst)
